# baseline (speedup 1.0000x reference)
import functools

import jax
import jax.numpy as jnp
from jax.experimental import pallas as pl
from jax.experimental.pallas import tpu as pltpu

LAMBDA_SOFTMAX = 9.0
LAMBDA_LSE = 6.0
MARGIN = 0.2
EPS = 1e-8

_INTERPRET = False


def _wnorm_kernel(s_ref, o_ref):
    x = s_ref[...]
    o_ref[...] = jnp.sqrt(jnp.sum(x * x, axis=1, keepdims=True))


def _scores_kernel(im_ref, capT_ref, mask_ref, w1_ref, E_ref, ET_ref, o_ref):
    imr = im_ref[0]
    mask = mask_ref[...]
    raw = jnp.dot(imr, capT_ref[...], preferred_element_type=jnp.float32)
    G = jax.lax.dot_general(imr, imr, (((1,), (1,)), ((), ())),
                            preferred_element_type=jnp.float32)
    lk = jnp.where(raw >= 0, raw, 0.1 * raw) * mask
    nsum = jnp.dot(lk * lk, E_ref[...], preferred_element_type=jnp.float32)
    ninv = 1.0 / (jnp.sqrt(nsum) + EPS)
    den = jnp.dot(ninv, ET_ref[...], preferred_element_type=jnp.float32)
    x = lk * den * LAMBDA_SOFTMAX
    m = jnp.max(x, axis=0, keepdims=True)
    e = jnp.exp(x - m)
    ssum = jnp.sum(e, axis=0, keepdims=True)
    a = e * (1.0 / ssum)
    w12 = jnp.sum(a * raw, axis=0, keepdims=True)
    v = jnp.dot(G, a, preferred_element_type=jnp.float32)
    w2 = jnp.sqrt(jnp.sum(a * v, axis=0, keepdims=True))
    sim = w12 / jnp.maximum(w1_ref[...] * w2, EPS)
    expd = jnp.exp(sim * LAMBDA_LSE) * mask
    ssc = jnp.dot(expd, E_ref[...], preferred_element_type=jnp.float32)
    o_ref[0, 0] = jnp.log(ssc) / LAMBDA_LSE


def _loss_kernel(sc_ref, o_ref):
    sc = sc_ref[...]
    B = sc.shape[0]
    ri = jax.lax.broadcasted_iota(jnp.int32, (B, B), 0)
    ci = jax.lax.broadcasted_iota(jnp.int32, (B, B), 1)
    eye = ri == ci
    diag_col = jnp.sum(jnp.where(eye, sc, 0.0), axis=1, keepdims=True)
    diag_row = jnp.sum(jnp.where(eye, sc, 0.0), axis=0, keepdims=True)
    cs = jnp.maximum(MARGIN + sc - diag_col, 0.0)
    cim = jnp.maximum(MARGIN + sc - diag_row, 0.0)
    cs = jnp.where(eye, 0.0, cs)
    cim = jnp.where(eye, 0.0, cim)
    s1 = jnp.sum(jnp.max(cs, axis=1, keepdims=True), axis=0, keepdims=True)
    s2 = jnp.sum(jnp.max(cim, axis=0, keepdims=True), axis=1, keepdims=True)
    o_ref[...] = s1 + s2


@functools.partial(jax.jit, static_argnames=())
def kernel(im, im_l, s, s_l):
    B, R, D = im.shape
    W = s.shape[1]
    NJ = 2
    CJ = B // NJ
    LJ = CJ * W

    s_flat = s.reshape(B * W, D)

    GW = 8
    w1_col = pl.pallas_call(
        _wnorm_kernel,
        grid=(GW,),
        in_specs=[pl.BlockSpec((B * W // GW, D), lambda g: (g, 0))],
        out_specs=pl.BlockSpec((B * W // GW, 1), lambda g: (g, 0)),
        out_shape=jax.ShapeDtypeStruct((B * W, 1), jnp.float32),
        name="word_norms",
        interpret=_INTERPRET,
    )(s_flat)
    w1_flat = w1_col.reshape(1, B * W)

    capT = s_flat.T.astype(jnp.bfloat16)
    im16 = im.astype(jnp.bfloat16)
    wpos = jnp.tile(jnp.arange(W, dtype=jnp.int32), B)
    slv = jnp.repeat(s_l.astype(jnp.int32), W)
    mask_flat = (wpos < slv).astype(jnp.float32).reshape(1, B * W)
    E = (jnp.arange(LJ, dtype=jnp.int32)[:, None] // W
         == jnp.arange(CJ, dtype=jnp.int32)[None, :]).astype(jnp.float32)
    ET = E.T

    scores4 = pl.pallas_call(
        _scores_kernel,
        grid=(NJ, B),
        in_specs=[
            pl.BlockSpec((1, R, D), lambda j, i: (i, 0, 0)),
            pl.BlockSpec((D, LJ), lambda j, i: (0, j)),
            pl.BlockSpec((1, LJ), lambda j, i: (0, j)),
            pl.BlockSpec((1, LJ), lambda j, i: (0, j)),
            pl.BlockSpec((LJ, CJ), lambda j, i: (0, 0)),
            pl.BlockSpec((CJ, LJ), lambda j, i: (0, 0)),
        ],
        out_specs=pl.BlockSpec((1, 1, 1, CJ), lambda j, i: (j, i, 0, 0)),
        out_shape=jax.ShapeDtypeStruct((NJ, B, 1, CJ), jnp.float32),
        compiler_params=pltpu.CompilerParams(
            dimension_semantics=("parallel", "arbitrary"),
            vmem_limit_bytes=56 * 1024 * 1024,
        ),
        name="caption_scores",
        interpret=_INTERPRET,
    )(im16, capT, mask_flat, w1_flat, E, ET)

    scores = scores4.reshape(NJ, B, CJ).transpose(1, 0, 2).reshape(B, B)

    loss2 = pl.pallas_call(
        _loss_kernel,
        out_shape=jax.ShapeDtypeStruct((1, 1), jnp.float32),
        name="hinge_loss",
        interpret=_INTERPRET,
    )(scores)
    return loss2.reshape(())


# speedup vs baseline: 1.9090x; 1.9090x over previous
import functools

import jax
import jax.numpy as jnp
from jax.experimental import pallas as pl
from jax.experimental.pallas import tpu as pltpu

LAMBDA_SOFTMAX = 9.0
LAMBDA_LSE = 6.0
MARGIN = 0.2
EPS = 1e-8

_INTERPRET = False

BI = 4
RP = 40


def _wnorm_kernel(s_ref, o_ref):
    x = s_ref[...]
    o_ref[...] = jnp.sqrt(jnp.sum(x * x, axis=1, keepdims=True))


def _scores_kernel(R, im_ref, capT_ref, mask_ref, w1_ref, E_ref, ET_ref,
                   o_ref, G_ref):
    imr = im_ref[0]
    M = BI * RP
    mask = mask_ref[...]
    raw = jnp.dot(imr, capT_ref[...], preferred_element_type=jnp.float32)
    G_ref[...] = jnp.zeros_like(G_ref)
    for bi in range(BI):
        sl = slice(bi * RP, (bi + 1) * RP)
        G_ref[sl, sl] = jax.lax.dot_general(
            imr[sl], imr[sl], (((1,), (1,)), ((), ())),
            preferred_element_type=jnp.float32)
    lk = jnp.where(raw >= 0, raw, 0.1 * raw) * mask
    nsum = jnp.dot(lk * lk, E_ref[...], preferred_element_type=jnp.float32)
    ninv = 1.0 / (jnp.sqrt(nsum) + EPS)
    den = jnp.dot(ninv, ET_ref[...], preferred_element_type=jnp.float32)
    x = lk * den * LAMBDA_SOFTMAX
    ri = jax.lax.broadcasted_iota(jnp.int32, (M, 1), 0)
    x = jnp.where(ri % RP < R, x, -1e30)
    x3 = x.reshape(BI, RP, -1)
    e3 = jnp.exp(x3)
    ssum3 = jnp.sum(e3, axis=1, keepdims=True)
    a3 = e3 * (1.0 / ssum3)
    a = a3.reshape(M, -1)
    w12 = jnp.sum(a3 * raw.reshape(BI, RP, -1), axis=1)
    v = jnp.dot(G_ref[...], a, preferred_element_type=jnp.float32)
    w2 = jnp.sqrt(jnp.sum(a3 * v.reshape(BI, RP, -1), axis=1))
    sim = w12 / jnp.maximum(w1_ref[...] * w2, EPS)
    expd = jnp.exp(sim * LAMBDA_LSE) * mask
    ssc = jnp.dot(expd, E_ref[...], preferred_element_type=jnp.float32)
    o_ref[0, 0] = jnp.log(ssc) / LAMBDA_LSE


def _loss_kernel(sc_ref, o_ref):
    sc = sc_ref[...]
    B = sc.shape[0]
    ri = jax.lax.broadcasted_iota(jnp.int32, (B, B), 0)
    ci = jax.lax.broadcasted_iota(jnp.int32, (B, B), 1)
    eye = ri == ci
    diag_col = jnp.sum(jnp.where(eye, sc, 0.0), axis=1, keepdims=True)
    diag_row = jnp.sum(jnp.where(eye, sc, 0.0), axis=0, keepdims=True)
    cs = jnp.maximum(MARGIN + sc - diag_col, 0.0)
    cim = jnp.maximum(MARGIN + sc - diag_row, 0.0)
    cs = jnp.where(eye, 0.0, cs)
    cim = jnp.where(eye, 0.0, cim)
    s1 = jnp.sum(jnp.max(cs, axis=1, keepdims=True), axis=0, keepdims=True)
    s2 = jnp.sum(jnp.max(cim, axis=0, keepdims=True), axis=1, keepdims=True)
    o_ref[...] = s1 + s2


@functools.partial(jax.jit, static_argnames=())
def kernel(im, im_l, s, s_l):
    B, R, D = im.shape
    W = s.shape[1]
    NJ = 2
    CJ = B // NJ
    LJ = CJ * W
    NB = B // BI

    s_flat = s.reshape(B * W, D)

    GW = 8
    w1_col = pl.pallas_call(
        _wnorm_kernel,
        grid=(GW,),
        in_specs=[pl.BlockSpec((B * W // GW, D), lambda g: (g, 0))],
        out_specs=pl.BlockSpec((B * W // GW, 1), lambda g: (g, 0)),
        out_shape=jax.ShapeDtypeStruct((B * W, 1), jnp.float32),
        name="word_norms",
        interpret=_INTERPRET,
    )(s_flat)
    w1_flat = w1_col.reshape(1, B * W)

    capT = s_flat.T.astype(jnp.bfloat16)
    im16 = jnp.pad(im, ((0, 0), (0, RP - R), (0, 0))).astype(jnp.bfloat16)
    im16 = im16.reshape(NB, BI * RP, D)
    wpos = jnp.tile(jnp.arange(W, dtype=jnp.int32), B)
    slv = jnp.repeat(s_l.astype(jnp.int32), W)
    mask_flat = (wpos < slv).astype(jnp.float32).reshape(1, B * W)
    E = (jnp.arange(LJ, dtype=jnp.int32)[:, None] // W
         == jnp.arange(CJ, dtype=jnp.int32)[None, :]).astype(jnp.float32)
    ET = E.T

    scores4 = pl.pallas_call(
        functools.partial(_scores_kernel, R),
        grid=(NJ, NB),
        in_specs=[
            pl.BlockSpec((1, BI * RP, D), lambda j, i: (i, 0, 0)),
            pl.BlockSpec((D, LJ), lambda j, i: (0, j)),
            pl.BlockSpec((1, LJ), lambda j, i: (0, j)),
            pl.BlockSpec((1, LJ), lambda j, i: (0, j)),
            pl.BlockSpec((LJ, CJ), lambda j, i: (0, 0)),
            pl.BlockSpec((CJ, LJ), lambda j, i: (0, 0)),
        ],
        out_specs=pl.BlockSpec((1, 1, BI, CJ), lambda j, i: (j, i, 0, 0)),
        out_shape=jax.ShapeDtypeStruct((NJ, NB, BI, CJ), jnp.float32),
        scratch_shapes=[pltpu.VMEM((BI * RP, BI * RP), jnp.float32)],
        compiler_params=pltpu.CompilerParams(
            dimension_semantics=("parallel", "arbitrary"),
            vmem_limit_bytes=56 * 1024 * 1024,
        ),
        name="caption_scores",
        interpret=_INTERPRET,
    )(im16, capT, mask_flat, w1_flat, E, ET)

    scores = scores4.reshape(NJ, B, CJ).transpose(1, 0, 2).reshape(B, B)

    loss2 = pl.pallas_call(
        _loss_kernel,
        out_shape=jax.ShapeDtypeStruct((1, 1), jnp.float32),
        name="hinge_loss",
        interpret=_INTERPRET,
    )(scores)
    return loss2.reshape(())


# speedup vs baseline: 2.0151x; 1.0556x over previous
import functools

import jax
import jax.numpy as jnp
from jax.experimental import pallas as pl
from jax.experimental.pallas import tpu as pltpu

LAMBDA_SOFTMAX = 9.0
LAMBDA_LSE = 6.0
MARGIN = 0.2
EPS = 1e-8

_INTERPRET = False

BI = 8
RP = 40


def _wnorm_kernel(s_ref, o_ref):
    x = s_ref[...]
    o_ref[...] = jnp.sqrt(jnp.sum(x * x, axis=1, keepdims=True))


def _scores_kernel(R, im_ref, capT_ref, mask_ref, w1_ref, E_ref, ET_ref,
                   o_ref, G_ref):
    imr = im_ref[0]
    M = BI * RP
    mask = mask_ref[...]
    raw = jnp.dot(imr, capT_ref[...], preferred_element_type=jnp.float32)
    G_ref[...] = jnp.zeros_like(G_ref)
    for bi in range(BI):
        sl = slice(bi * RP, (bi + 1) * RP)
        G_ref[sl, sl] = jax.lax.dot_general(
            imr[sl], imr[sl], (((1,), (1,)), ((), ())),
            preferred_element_type=jnp.float32)
    lk = jnp.where(raw >= 0, raw, 0.1 * raw) * mask
    nsum = jnp.dot(lk * lk, E_ref[...], preferred_element_type=jnp.float32)
    ninv = 1.0 / (jnp.sqrt(nsum) + EPS)
    den = jnp.dot(ninv, ET_ref[...], preferred_element_type=jnp.float32)
    x = lk * den * LAMBDA_SOFTMAX
    ri = jax.lax.broadcasted_iota(jnp.int32, (M, 1), 0)
    x = jnp.where(ri % RP < R, x, -1e30)
    x3 = x.reshape(BI, RP, -1)
    e3 = jnp.exp(x3)
    ssum3 = jnp.sum(e3, axis=1, keepdims=True)
    a3 = e3 * (1.0 / ssum3)
    a = a3.reshape(M, -1)
    w12 = jnp.sum(a3 * raw.reshape(BI, RP, -1), axis=1)
    v = jnp.dot(G_ref[...], a, preferred_element_type=jnp.float32)
    w2 = jnp.sqrt(jnp.sum(a3 * v.reshape(BI, RP, -1), axis=1))
    sim = w12 / jnp.maximum(w1_ref[...] * w2, EPS)
    expd = jnp.exp(sim * LAMBDA_LSE) * mask
    ssc = jnp.dot(expd, E_ref[...], preferred_element_type=jnp.float32)
    o_ref[0, 0] = jnp.log(ssc) / LAMBDA_LSE


def _loss_kernel(sc_ref, o_ref):
    sc = sc_ref[...]
    B = sc.shape[0]
    ri = jax.lax.broadcasted_iota(jnp.int32, (B, B), 0)
    ci = jax.lax.broadcasted_iota(jnp.int32, (B, B), 1)
    eye = ri == ci
    diag_col = jnp.sum(jnp.where(eye, sc, 0.0), axis=1, keepdims=True)
    diag_row = jnp.sum(jnp.where(eye, sc, 0.0), axis=0, keepdims=True)
    cs = jnp.maximum(MARGIN + sc - diag_col, 0.0)
    cim = jnp.maximum(MARGIN + sc - diag_row, 0.0)
    cs = jnp.where(eye, 0.0, cs)
    cim = jnp.where(eye, 0.0, cim)
    s1 = jnp.sum(jnp.max(cs, axis=1, keepdims=True), axis=0, keepdims=True)
    s2 = jnp.sum(jnp.max(cim, axis=0, keepdims=True), axis=1, keepdims=True)
    o_ref[...] = s1 + s2


@functools.partial(jax.jit, static_argnames=())
def kernel(im, im_l, s, s_l):
    B, R, D = im.shape
    W = s.shape[1]
    NJ = 2
    CJ = B // NJ
    LJ = CJ * W
    NB = B // BI

    s_flat = s.reshape(B * W, D)

    GW = 8
    w1_col = pl.pallas_call(
        _wnorm_kernel,
        grid=(GW,),
        in_specs=[pl.BlockSpec((B * W // GW, D), lambda g: (g, 0))],
        out_specs=pl.BlockSpec((B * W // GW, 1), lambda g: (g, 0)),
        out_shape=jax.ShapeDtypeStruct((B * W, 1), jnp.float32),
        name="word_norms",
        interpret=_INTERPRET,
    )(s_flat)
    w1_flat = w1_col.reshape(1, B * W)

    capT = s_flat.T.astype(jnp.bfloat16)
    im16 = jnp.pad(im, ((0, 0), (0, RP - R), (0, 0))).astype(jnp.bfloat16)
    im16 = im16.reshape(NB, BI * RP, D)
    wpos = jnp.tile(jnp.arange(W, dtype=jnp.int32), B)
    slv = jnp.repeat(s_l.astype(jnp.int32), W)
    mask_flat = (wpos < slv).astype(jnp.float32).reshape(1, B * W)
    E = (jnp.arange(LJ, dtype=jnp.int32)[:, None] // W
         == jnp.arange(CJ, dtype=jnp.int32)[None, :]).astype(jnp.float32)
    ET = E.T

    scores4 = pl.pallas_call(
        functools.partial(_scores_kernel, R),
        grid=(NJ, NB),
        in_specs=[
            pl.BlockSpec((1, BI * RP, D), lambda j, i: (i, 0, 0)),
            pl.BlockSpec((D, LJ), lambda j, i: (0, j)),
            pl.BlockSpec((1, LJ), lambda j, i: (0, j)),
            pl.BlockSpec((1, LJ), lambda j, i: (0, j)),
            pl.BlockSpec((LJ, CJ), lambda j, i: (0, 0)),
            pl.BlockSpec((CJ, LJ), lambda j, i: (0, 0)),
        ],
        out_specs=pl.BlockSpec((1, 1, BI, CJ), lambda j, i: (j, i, 0, 0)),
        out_shape=jax.ShapeDtypeStruct((NJ, NB, BI, CJ), jnp.float32),
        scratch_shapes=[pltpu.VMEM((BI * RP, BI * RP), jnp.float32)],
        compiler_params=pltpu.CompilerParams(
            dimension_semantics=("parallel", "arbitrary"),
            vmem_limit_bytes=56 * 1024 * 1024,
        ),
        name="caption_scores",
        interpret=_INTERPRET,
    )(im16, capT, mask_flat, w1_flat, E, ET)

    scores = scores4.reshape(NJ, B, CJ).transpose(1, 0, 2).reshape(B, B)

    loss2 = pl.pallas_call(
        _loss_kernel,
        out_shape=jax.ShapeDtypeStruct((1, 1), jnp.float32),
        name="hinge_loss",
        interpret=_INTERPRET,
    )(scores)
    return loss2.reshape(())


# speedup vs baseline: 2.1117x; 1.0480x over previous
import functools

import jax
import jax.numpy as jnp
from jax.experimental import pallas as pl
from jax.experimental.pallas import tpu as pltpu

LAMBDA_SOFTMAX = 9.0
LAMBDA_LSE = 6.0
MARGIN = 0.2
EPS = 1e-8

_INTERPRET = False

BI = 8
RP = 40


def _wnorm_kernel(s_ref, o_ref, t_ref):
    x = s_ref[...]
    o_ref[...] = jnp.sqrt(jnp.sum(x * x, axis=1, keepdims=True))
    t_ref[...] = x.astype(jnp.bfloat16).T


def _scores_kernel(R, im_ref, capT_ref, mask_ref, w1_ref, E_ref, ET_ref,
                   o_ref, G_ref):
    imr = im_ref[0]
    M = BI * RP
    mask = mask_ref[...]
    raw = jnp.dot(imr, capT_ref[...], preferred_element_type=jnp.float32)
    G_ref[...] = jnp.zeros_like(G_ref)
    for bi in range(BI):
        sl = slice(bi * RP, (bi + 1) * RP)
        G_ref[sl, sl] = jax.lax.dot_general(
            imr[sl], imr[sl], (((1,), (1,)), ((), ())),
            preferred_element_type=jnp.float32)
    lk = jnp.where(raw >= 0, raw, 0.1 * raw) * mask
    nsum = jnp.dot(lk * lk, E_ref[...], preferred_element_type=jnp.float32)
    ninv = 1.0 / (jnp.sqrt(nsum) + EPS)
    den = jnp.dot(ninv, ET_ref[...], preferred_element_type=jnp.float32)
    x = lk * den * LAMBDA_SOFTMAX
    ri = jax.lax.broadcasted_iota(jnp.int32, (M, 1), 0)
    x = jnp.where(ri % RP < R, x, -1e30)
    x3 = x.reshape(BI, RP, -1)
    e3 = jnp.exp(x3)
    ssum3 = jnp.sum(e3, axis=1, keepdims=True)
    a3 = e3 * (1.0 / ssum3)
    a = a3.reshape(M, -1)
    w12 = jnp.sum(a3 * raw.reshape(BI, RP, -1), axis=1)
    v = jnp.dot(G_ref[...], a, preferred_element_type=jnp.float32)
    w2 = jnp.sqrt(jnp.sum(a3 * v.reshape(BI, RP, -1), axis=1))
    sim = w12 / jnp.maximum(w1_ref[...] * w2, EPS)
    expd = jnp.exp(sim * LAMBDA_LSE) * mask
    ssc = jnp.dot(expd, E_ref[...], preferred_element_type=jnp.float32)
    o_ref[0, 0] = jnp.log(ssc) / LAMBDA_LSE


def _loss_kernel(sc_ref, o_ref):
    sc = sc_ref[...]
    B = sc.shape[0]
    ri = jax.lax.broadcasted_iota(jnp.int32, (B, B), 0)
    ci = jax.lax.broadcasted_iota(jnp.int32, (B, B), 1)
    eye = ri == ci
    diag_col = jnp.sum(jnp.where(eye, sc, 0.0), axis=1, keepdims=True)
    diag_row = jnp.sum(jnp.where(eye, sc, 0.0), axis=0, keepdims=True)
    cs = jnp.maximum(MARGIN + sc - diag_col, 0.0)
    cim = jnp.maximum(MARGIN + sc - diag_row, 0.0)
    cs = jnp.where(eye, 0.0, cs)
    cim = jnp.where(eye, 0.0, cim)
    s1 = jnp.sum(jnp.max(cs, axis=1, keepdims=True), axis=0, keepdims=True)
    s2 = jnp.sum(jnp.max(cim, axis=0, keepdims=True), axis=1, keepdims=True)
    o_ref[...] = s1 + s2


@functools.partial(jax.jit, static_argnames=())
def kernel(im, im_l, s, s_l):
    B, R, D = im.shape
    W = s.shape[1]
    NJ = 2
    CJ = B // NJ
    LJ = CJ * W
    NB = B // BI

    s_flat = s.reshape(B * W, D)

    GW = 5
    w1_col, capT = pl.pallas_call(
        _wnorm_kernel,
        grid=(GW,),
        in_specs=[pl.BlockSpec((B * W // GW, D), lambda g: (g, 0))],
        out_specs=[
            pl.BlockSpec((B * W // GW, 1), lambda g: (g, 0)),
            pl.BlockSpec((D, B * W // GW), lambda g: (0, g)),
        ],
        out_shape=[
            jax.ShapeDtypeStruct((B * W, 1), jnp.float32),
            jax.ShapeDtypeStruct((D, B * W), jnp.bfloat16),
        ],
        name="word_norms_capT",
        interpret=_INTERPRET,
    )(s_flat)
    w1_flat = w1_col.reshape(1, B * W)
    im16 = jnp.pad(im, ((0, 0), (0, RP - R), (0, 0))).astype(jnp.bfloat16)
    im16 = im16.reshape(NB, BI * RP, D)
    wpos = jnp.tile(jnp.arange(W, dtype=jnp.int32), B)
    slv = jnp.repeat(s_l.astype(jnp.int32), W)
    mask_flat = (wpos < slv).astype(jnp.float32).reshape(1, B * W)
    E = (jnp.arange(LJ, dtype=jnp.int32)[:, None] // W
         == jnp.arange(CJ, dtype=jnp.int32)[None, :]).astype(jnp.float32)
    ET = E.T

    scores4 = pl.pallas_call(
        functools.partial(_scores_kernel, R),
        grid=(NJ, NB),
        in_specs=[
            pl.BlockSpec((1, BI * RP, D), lambda j, i: (i, 0, 0)),
            pl.BlockSpec((D, LJ), lambda j, i: (0, j)),
            pl.BlockSpec((1, LJ), lambda j, i: (0, j)),
            pl.BlockSpec((1, LJ), lambda j, i: (0, j)),
            pl.BlockSpec((LJ, CJ), lambda j, i: (0, 0)),
            pl.BlockSpec((CJ, LJ), lambda j, i: (0, 0)),
        ],
        out_specs=pl.BlockSpec((1, 1, BI, CJ), lambda j, i: (j, i, 0, 0)),
        out_shape=jax.ShapeDtypeStruct((NJ, NB, BI, CJ), jnp.float32),
        scratch_shapes=[pltpu.VMEM((BI * RP, BI * RP), jnp.float32)],
        compiler_params=pltpu.CompilerParams(
            dimension_semantics=("parallel", "arbitrary"),
            vmem_limit_bytes=56 * 1024 * 1024,
        ),
        name="caption_scores",
        interpret=_INTERPRET,
    )(im16, capT, mask_flat, w1_flat, E, ET)

    scores = scores4.reshape(NJ, B, CJ).transpose(1, 0, 2).reshape(B, B)

    loss2 = pl.pallas_call(
        _loss_kernel,
        out_shape=jax.ShapeDtypeStruct((1, 1), jnp.float32),
        name="hinge_loss",
        interpret=_INTERPRET,
    )(scores)
    return loss2.reshape(())


# speedup vs baseline: 2.2319x; 1.0569x over previous
import functools

import jax
import jax.numpy as jnp
from jax.experimental import pallas as pl
from jax.experimental.pallas import tpu as pltpu

LAMBDA_SOFTMAX = 9.0
LAMBDA_LSE = 6.0
MARGIN = 0.2
EPS = 1e-8

_INTERPRET = False

BI = 8
RP = 40


def _prep_kernel(s_ref, o_ref, t_ref):
    x = s_ref[...]
    o_ref[...] = jnp.sqrt(jnp.sum(x * x, axis=1, keepdims=True)).T
    t_ref[...] = x.astype(jnp.bfloat16).T


def _scores_kernel(R, im_ref, capT_ref, mask_ref, w1_ref, E_ref, ET_ref,
                   o_ref, imp_ref, G_ref):
    first = (pl.program_id(0) == 0) & (pl.program_id(1) == 0)

    @pl.when(first)
    def _():
        imp_ref[...] = jnp.zeros_like(imp_ref)
        G_ref[...] = jnp.zeros_like(G_ref)

    for bi in range(BI):
        imp_ref[bi * RP:bi * RP + R, :] = im_ref[bi].astype(jnp.bfloat16)
    imr = imp_ref[...]
    M = BI * RP
    mask = mask_ref[...]
    raw = jnp.dot(imr, capT_ref[...], preferred_element_type=jnp.float32)
    for bi in range(BI):
        sl = slice(bi * RP, (bi + 1) * RP)
        G_ref[sl, sl] = jax.lax.dot_general(
            imr[sl], imr[sl], (((1,), (1,)), ((), ())),
            preferred_element_type=jnp.float32)
    lk = jnp.where(raw >= 0, raw, 0.1 * raw) * mask
    nsum = jnp.dot(lk * lk, E_ref[...], preferred_element_type=jnp.float32)
    ninv = 1.0 / (jnp.sqrt(nsum) + EPS)
    den = jnp.dot(ninv, ET_ref[...], preferred_element_type=jnp.float32)
    x = lk * den * LAMBDA_SOFTMAX
    ri = jax.lax.broadcasted_iota(jnp.int32, (M, 1), 0)
    x = jnp.where(ri % RP < R, x, -1e30)
    x3 = x.reshape(BI, RP, -1)
    e3 = jnp.exp(x3)
    ssum3 = jnp.sum(e3, axis=1, keepdims=True)
    a3 = e3 * (1.0 / ssum3)
    a = a3.reshape(M, -1)
    w12 = jnp.sum(a3 * raw.reshape(BI, RP, -1), axis=1)
    v = jnp.dot(G_ref[...], a, preferred_element_type=jnp.float32)
    w2 = jnp.sqrt(jnp.sum(a3 * v.reshape(BI, RP, -1), axis=1))
    sim = w12 / jnp.maximum(w1_ref[...] * w2, EPS)
    expd = jnp.exp(sim * LAMBDA_LSE) * mask
    ssc = jnp.dot(expd, E_ref[...], preferred_element_type=jnp.float32)
    o_ref[0, 0] = jnp.log(ssc) / LAMBDA_LSE


def _loss_kernel(sc_ref, o_ref):
    s4 = sc_ref[...]
    NJ = s4.shape[0]
    B = s4.shape[1] * s4.shape[2]
    s3 = s4.reshape(NJ, B, s4.shape[3])
    sc = jnp.concatenate([s3[j] for j in range(NJ)], axis=1)
    ri = jax.lax.broadcasted_iota(jnp.int32, (B, B), 0)
    ci = jax.lax.broadcasted_iota(jnp.int32, (B, B), 1)
    eye = ri == ci
    diag_col = jnp.sum(jnp.where(eye, sc, 0.0), axis=1, keepdims=True)
    diag_row = jnp.sum(jnp.where(eye, sc, 0.0), axis=0, keepdims=True)
    cs = jnp.maximum(MARGIN + sc - diag_col, 0.0)
    cim = jnp.maximum(MARGIN + sc - diag_row, 0.0)
    cs = jnp.where(eye, 0.0, cs)
    cim = jnp.where(eye, 0.0, cim)
    s1 = jnp.sum(jnp.max(cs, axis=1, keepdims=True), axis=0, keepdims=True)
    s2 = jnp.sum(jnp.max(cim, axis=0, keepdims=True), axis=1, keepdims=True)
    o_ref[...] = s1 + s2


@functools.partial(jax.jit, static_argnames=())
def kernel(im, im_l, s, s_l):
    B, R, D = im.shape
    W = s.shape[1]
    NJ = 2
    CJ = B // NJ
    LJ = CJ * W
    NB = B // BI

    s_flat = s.reshape(B * W, D)

    GW = 5
    w1_flat, capT = pl.pallas_call(
        _prep_kernel,
        grid=(GW,),
        in_specs=[pl.BlockSpec((B * W // GW, D), lambda g: (g, 0))],
        out_specs=[
            pl.BlockSpec((1, B * W // GW), lambda g: (0, g)),
            pl.BlockSpec((D, B * W // GW), lambda g: (0, g)),
        ],
        out_shape=[
            jax.ShapeDtypeStruct((1, B * W), jnp.float32),
            jax.ShapeDtypeStruct((D, B * W), jnp.bfloat16),
        ],
        name="word_norms_capT",
        interpret=_INTERPRET,
    )(s_flat)

    wpos = jnp.tile(jnp.arange(W, dtype=jnp.int32), B)
    slv = jnp.repeat(s_l.astype(jnp.int32), W)
    mask_flat = (wpos < slv).astype(jnp.float32).reshape(1, B * W)

    E = (jnp.arange(LJ, dtype=jnp.int32)[:, None] // W
         == jnp.arange(CJ, dtype=jnp.int32)[None, :]).astype(jnp.float32)
    ET = E.T

    scores4 = pl.pallas_call(
        functools.partial(_scores_kernel, R),
        grid=(NJ, NB),
        in_specs=[
            pl.BlockSpec((BI, R, D), lambda j, i: (i, 0, 0)),
            pl.BlockSpec((D, LJ), lambda j, i: (0, j)),
            pl.BlockSpec((1, LJ), lambda j, i: (0, j)),
            pl.BlockSpec((1, LJ), lambda j, i: (0, j)),
            pl.BlockSpec((LJ, CJ), lambda j, i: (0, 0)),
            pl.BlockSpec((CJ, LJ), lambda j, i: (0, 0)),
        ],
        out_specs=pl.BlockSpec((1, 1, BI, CJ), lambda j, i: (j, i, 0, 0)),
        out_shape=jax.ShapeDtypeStruct((NJ, NB, BI, CJ), jnp.float32),
        scratch_shapes=[
            pltpu.VMEM((BI * RP, D), jnp.bfloat16),
            pltpu.VMEM((BI * RP, BI * RP), jnp.float32),
        ],
        compiler_params=pltpu.CompilerParams(
            dimension_semantics=("parallel", "arbitrary"),
            vmem_limit_bytes=56 * 1024 * 1024,
        ),
        name="caption_scores",
        interpret=_INTERPRET,
    )(im, capT, mask_flat, w1_flat, E, ET)

    loss2 = pl.pallas_call(
        _loss_kernel,
        out_shape=jax.ShapeDtypeStruct((1, 1), jnp.float32),
        name="hinge_loss",
        interpret=_INTERPRET,
    )(scores4)
    return loss2.reshape(())


# speedup vs baseline: 2.2639x; 1.0143x over previous
import functools

import jax
import jax.numpy as jnp
from jax.experimental import pallas as pl
from jax.experimental.pallas import tpu as pltpu

LAMBDA_SOFTMAX = 9.0
LAMBDA_LSE = 6.0
MARGIN = 0.2
EPS = 1e-8

_INTERPRET = False

BI = 8
RP = 40


def _prep_kernel(s_ref, o_ref, t_ref):
    x = s_ref[...]
    o_ref[...] = jnp.sqrt(jnp.sum(x * x, axis=1, keepdims=True)).T
    t_ref[...] = x.astype(jnp.bfloat16).T


def _scores_kernel(R, im_ref, capT_ref, mask_ref, w1_ref, E_ref, ET_ref,
                   o_ref, imp_ref, G_ref):
    first = (pl.program_id(0) == 0) & (pl.program_id(1) == 0)

    @pl.when(first)
    def _():
        imp_ref[...] = jnp.zeros_like(imp_ref)
        G_ref[...] = jnp.zeros_like(G_ref)

    for bi in range(BI):
        imp_ref[bi * RP:bi * RP + R, :] = im_ref[bi].astype(jnp.bfloat16)
    imr = imp_ref[...]
    M = BI * RP
    mask = mask_ref[...]
    raw = jnp.dot(imr, capT_ref[...], preferred_element_type=jnp.float32)
    for bi in range(BI):
        sl = slice(bi * RP, (bi + 1) * RP)
        G_ref[sl, sl] = jax.lax.dot_general(
            imr[sl], imr[sl], (((1,), (1,)), ((), ())),
            preferred_element_type=jnp.float32)
    lk = jnp.where(raw >= 0, raw, 0.1 * raw) * mask
    nsum = jnp.dot(lk * lk, E_ref[...], preferred_element_type=jnp.float32)
    ninv = LAMBDA_SOFTMAX / (jnp.sqrt(nsum) + EPS)
    den = jnp.dot(ninv, ET_ref[...], preferred_element_type=jnp.float32)
    x = lk * den
    ri = jax.lax.broadcasted_iota(jnp.int32, (M, 1), 0)
    x = jnp.where(ri % RP < R, x, -1e30)
    e = jnp.exp(x)
    e3 = e.reshape(BI, RP, -1)
    ssum = jnp.sum(e3, axis=1)
    S12 = jnp.sum(e3 * raw.reshape(BI, RP, -1), axis=1)
    v = jnp.dot(G_ref[...], e, preferred_element_type=jnp.float32)
    S2 = jnp.sum(e3 * v.reshape(BI, RP, -1), axis=1)
    sim = S12 / jnp.maximum(w1_ref[...] * jnp.sqrt(S2), EPS * ssum)
    expd = jnp.exp(sim * LAMBDA_LSE) * mask
    ssc = jnp.dot(expd, E_ref[...], preferred_element_type=jnp.float32)
    o_ref[0, 0] = jnp.log(ssc) / LAMBDA_LSE


def _loss_kernel(sc_ref, o_ref):
    s4 = sc_ref[...]
    NJ = s4.shape[0]
    B = s4.shape[1] * s4.shape[2]
    s3 = s4.reshape(NJ, B, s4.shape[3])
    sc = jnp.concatenate([s3[j] for j in range(NJ)], axis=1)
    ri = jax.lax.broadcasted_iota(jnp.int32, (B, B), 0)
    ci = jax.lax.broadcasted_iota(jnp.int32, (B, B), 1)
    eye = ri == ci
    diag_col = jnp.sum(jnp.where(eye, sc, 0.0), axis=1, keepdims=True)
    diag_row = jnp.sum(jnp.where(eye, sc, 0.0), axis=0, keepdims=True)
    cs = jnp.maximum(MARGIN + sc - diag_col, 0.0)
    cim = jnp.maximum(MARGIN + sc - diag_row, 0.0)
    cs = jnp.where(eye, 0.0, cs)
    cim = jnp.where(eye, 0.0, cim)
    s1 = jnp.sum(jnp.max(cs, axis=1, keepdims=True), axis=0, keepdims=True)
    s2 = jnp.sum(jnp.max(cim, axis=0, keepdims=True), axis=1, keepdims=True)
    o_ref[...] = s1 + s2


@functools.partial(jax.jit, static_argnames=())
def kernel(im, im_l, s, s_l):
    B, R, D = im.shape
    W = s.shape[1]
    NJ = 2
    CJ = B // NJ
    LJ = CJ * W
    NB = B // BI

    s_flat = s.reshape(B * W, D)

    GW = 5
    w1_flat, capT = pl.pallas_call(
        _prep_kernel,
        grid=(GW,),
        in_specs=[pl.BlockSpec((B * W // GW, D), lambda g: (g, 0))],
        out_specs=[
            pl.BlockSpec((1, B * W // GW), lambda g: (0, g)),
            pl.BlockSpec((D, B * W // GW), lambda g: (0, g)),
        ],
        out_shape=[
            jax.ShapeDtypeStruct((1, B * W), jnp.float32),
            jax.ShapeDtypeStruct((D, B * W), jnp.bfloat16),
        ],
        name="word_norms_capT",
        interpret=_INTERPRET,
    )(s_flat)

    wpos = jnp.tile(jnp.arange(W, dtype=jnp.int32), B)
    slv = jnp.repeat(s_l.astype(jnp.int32), W)
    mask_flat = (wpos < slv).astype(jnp.float32).reshape(1, B * W)

    E = (jnp.arange(LJ, dtype=jnp.int32)[:, None] // W
         == jnp.arange(CJ, dtype=jnp.int32)[None, :]).astype(jnp.float32)
    ET = E.T

    scores4 = pl.pallas_call(
        functools.partial(_scores_kernel, R),
        grid=(NJ, NB),
        in_specs=[
            pl.BlockSpec((BI, R, D), lambda j, i: (i, 0, 0)),
            pl.BlockSpec((D, LJ), lambda j, i: (0, j)),
            pl.BlockSpec((1, LJ), lambda j, i: (0, j)),
            pl.BlockSpec((1, LJ), lambda j, i: (0, j)),
            pl.BlockSpec((LJ, CJ), lambda j, i: (0, 0)),
            pl.BlockSpec((CJ, LJ), lambda j, i: (0, 0)),
        ],
        out_specs=pl.BlockSpec((1, 1, BI, CJ), lambda j, i: (j, i, 0, 0)),
        out_shape=jax.ShapeDtypeStruct((NJ, NB, BI, CJ), jnp.float32),
        scratch_shapes=[
            pltpu.VMEM((BI * RP, D), jnp.bfloat16),
            pltpu.VMEM((BI * RP, BI * RP), jnp.float32),
        ],
        compiler_params=pltpu.CompilerParams(
            dimension_semantics=("parallel", "arbitrary"),
            vmem_limit_bytes=56 * 1024 * 1024,
        ),
        name="caption_scores",
        interpret=_INTERPRET,
    )(im, capT, mask_flat, w1_flat, E, ET)

    loss2 = pl.pallas_call(
        _loss_kernel,
        out_shape=jax.ShapeDtypeStruct((1, 1), jnp.float32),
        name="hinge_loss",
        interpret=_INTERPRET,
    )(scores4)
    return loss2.reshape(())


# speedup vs baseline: 2.2644x; 1.0002x over previous
import functools

import jax
import jax.numpy as jnp
from jax.experimental import pallas as pl
from jax.experimental.pallas import tpu as pltpu

LAMBDA_SOFTMAX = 9.0
LAMBDA_LSE = 6.0
MARGIN = 0.2
EPS = 1e-8

_INTERPRET = False

BI = 8
RP = 40


def _prep_kernel(s_ref, o_ref, t_ref):
    x = s_ref[...]
    o_ref[...] = jnp.sqrt(jnp.sum(x * x, axis=1, keepdims=True)).T
    t_ref[...] = x.astype(jnp.bfloat16).T


def _scores_kernel(R, im_ref, capT_ref, mask_ref, w1_ref, E_ref, ET_ref,
                   o_ref, imp_ref, G_ref, expd_ref):
    first = (pl.program_id(0) == 0) & (pl.program_id(1) == 0)

    @pl.when(first)
    def _():
        imp_ref[...] = jnp.zeros_like(imp_ref)
        G_ref[...] = jnp.zeros_like(G_ref)

    for bi in range(BI):
        imp_ref[bi * RP:bi * RP + R, :] = im_ref[bi].astype(jnp.bfloat16)
    imr = imp_ref[...]
    mask = mask_ref[...]
    raw = jnp.dot(imr, capT_ref[...], preferred_element_type=jnp.float32)
    HB = BI // 2
    HM = HB * RP
    for bi in range(BI):
        g, l = divmod(bi, HB)
        sl = slice(l * RP, (l + 1) * RP)
        G_ref[g, sl, sl] = jax.lax.dot_general(
            imr[bi * RP:(bi + 1) * RP], imr[bi * RP:(bi + 1) * RP],
            (((1,), (1,)), ((), ())), preferred_element_type=jnp.float32)
    ri = jax.lax.broadcasted_iota(jnp.int32, (HM, 1), 0)
    rmask = ri % RP < R
    for g in range(2):
        rawg = raw[g * HM:(g + 1) * HM]
        lk = jnp.where(rawg >= 0, rawg, 0.1 * rawg) * mask
        nsum = jnp.dot(lk * lk, E_ref[...], preferred_element_type=jnp.float32)
        ninv = LAMBDA_SOFTMAX / (jnp.sqrt(nsum) + EPS)
        den = jnp.dot(ninv, ET_ref[...], preferred_element_type=jnp.float32)
        x = jnp.where(rmask, lk * den, -1e30)
        e = jnp.exp(x)
        e3 = e.reshape(HB, RP, -1)
        ssum = jnp.sum(e3, axis=1)
        S12 = jnp.sum(e3 * rawg.reshape(HB, RP, -1), axis=1)
        v = jnp.dot(G_ref[g], e, preferred_element_type=jnp.float32)
        S2 = jnp.sum(e3 * v.reshape(HB, RP, -1), axis=1)
        sim = S12 / jnp.maximum(w1_ref[...] * jnp.sqrt(S2), EPS * ssum)
        expd_ref[g * HB:(g + 1) * HB] = jnp.exp(sim * LAMBDA_LSE) * mask
    ssc = jnp.dot(expd_ref[...], E_ref[...],
                  preferred_element_type=jnp.float32)
    o_ref[0, 0] = jnp.log(ssc) / LAMBDA_LSE


def _loss_kernel(sc_ref, o_ref):
    s4 = sc_ref[...]
    NJ = s4.shape[0]
    B = s4.shape[1] * s4.shape[2]
    s3 = s4.reshape(NJ, B, s4.shape[3])
    sc = jnp.concatenate([s3[j] for j in range(NJ)], axis=1)
    ri = jax.lax.broadcasted_iota(jnp.int32, (B, B), 0)
    ci = jax.lax.broadcasted_iota(jnp.int32, (B, B), 1)
    eye = ri == ci
    diag_col = jnp.sum(jnp.where(eye, sc, 0.0), axis=1, keepdims=True)
    diag_row = jnp.sum(jnp.where(eye, sc, 0.0), axis=0, keepdims=True)
    cs = jnp.maximum(MARGIN + sc - diag_col, 0.0)
    cim = jnp.maximum(MARGIN + sc - diag_row, 0.0)
    cs = jnp.where(eye, 0.0, cs)
    cim = jnp.where(eye, 0.0, cim)
    s1 = jnp.sum(jnp.max(cs, axis=1, keepdims=True), axis=0, keepdims=True)
    s2 = jnp.sum(jnp.max(cim, axis=0, keepdims=True), axis=1, keepdims=True)
    o_ref[...] = s1 + s2


@functools.partial(jax.jit, static_argnames=())
def kernel(im, im_l, s, s_l):
    B, R, D = im.shape
    W = s.shape[1]
    NJ = 2
    CJ = B // NJ
    LJ = CJ * W
    NB = B // BI

    s_flat = s.reshape(B * W, D)

    GW = 5
    w1_flat, capT = pl.pallas_call(
        _prep_kernel,
        grid=(GW,),
        in_specs=[pl.BlockSpec((B * W // GW, D), lambda g: (g, 0))],
        out_specs=[
            pl.BlockSpec((1, B * W // GW), lambda g: (0, g)),
            pl.BlockSpec((D, B * W // GW), lambda g: (0, g)),
        ],
        out_shape=[
            jax.ShapeDtypeStruct((1, B * W), jnp.float32),
            jax.ShapeDtypeStruct((D, B * W), jnp.bfloat16),
        ],
        name="word_norms_capT",
        interpret=_INTERPRET,
    )(s_flat)

    wpos = jnp.tile(jnp.arange(W, dtype=jnp.int32), B)
    slv = jnp.repeat(s_l.astype(jnp.int32), W)
    mask_flat = (wpos < slv).astype(jnp.float32).reshape(1, B * W)

    E = (jnp.arange(LJ, dtype=jnp.int32)[:, None] // W
         == jnp.arange(CJ, dtype=jnp.int32)[None, :]).astype(jnp.float32)
    ET = E.T

    scores4 = pl.pallas_call(
        functools.partial(_scores_kernel, R),
        grid=(NJ, NB),
        in_specs=[
            pl.BlockSpec((BI, R, D), lambda j, i: (i, 0, 0)),
            pl.BlockSpec((D, LJ), lambda j, i: (0, j)),
            pl.BlockSpec((1, LJ), lambda j, i: (0, j)),
            pl.BlockSpec((1, LJ), lambda j, i: (0, j)),
            pl.BlockSpec((LJ, CJ), lambda j, i: (0, 0)),
            pl.BlockSpec((CJ, LJ), lambda j, i: (0, 0)),
        ],
        out_specs=pl.BlockSpec((1, 1, BI, CJ), lambda j, i: (j, i, 0, 0)),
        out_shape=jax.ShapeDtypeStruct((NJ, NB, BI, CJ), jnp.float32),
        scratch_shapes=[
            pltpu.VMEM((BI * RP, D), jnp.bfloat16),
            pltpu.VMEM((2, BI * RP // 2, BI * RP // 2), jnp.float32),
            pltpu.VMEM((BI, LJ), jnp.float32),
        ],
        compiler_params=pltpu.CompilerParams(
            dimension_semantics=("parallel", "arbitrary"),
            vmem_limit_bytes=56 * 1024 * 1024,
        ),
        name="caption_scores",
        interpret=_INTERPRET,
    )(im, capT, mask_flat, w1_flat, E, ET)

    loss2 = pl.pallas_call(
        _loss_kernel,
        out_shape=jax.ShapeDtypeStruct((1, 1), jnp.float32),
        name="hinge_loss",
        interpret=_INTERPRET,
    )(scores4)
    return loss2.reshape(())
